# baseline (speedup 1.0000x reference)
import functools

import jax
import jax.numpy as jnp
from jax import lax
from jax.experimental import pallas as pl
from jax.experimental.pallas import tpu as pltpu
from jax.experimental.pallas import tpu_sc as plsc

V = 1000
NFLAT = 51200
NC, NS, L = 2, 16, 16
NW = NC * NS
NVH = V // 8
NBAND = 4
SEG = 1024
NSEG = NFLAT // SEG
TPS = SEG // 128
NSTG = 2 * NBAND
ROWS_PER_TILE = NFLAT // NW


def _lse_body(tab_ref, lse_ref, tt_ref):
    x = tab_ref[...]
    m = jnp.max(x, axis=1, keepdims=True)
    s = jnp.sum(jnp.exp(x - m), axis=1, keepdims=True)
    lse_ref[...] = m + jnp.log(s)
    tt_ref[...] = x.T


def _table_prep(table):
    return pl.pallas_call(
        _lse_body,
        out_shape=[
            jax.ShapeDtypeStruct((V, 1), jnp.float32),
            jax.ShapeDtypeStruct((V, V), jnp.float32),
        ],
    )(table)


_mesh = plsc.VectorSubcoreMesh(core_axis_name="c", subcore_axis_name="s")


@functools.partial(
    pl.kernel,
    mesh=_mesh,
    compiler_params=pltpu.CompilerParams(
        use_tc_tiling_on_sc=False, needs_layout_passes=False),
    out_type=[
        jax.ShapeDtypeStruct((NVH, NFLAT // 128, 8, 128), jnp.float32),
        jax.ShapeDtypeStruct((NW, L), jnp.float32),
    ],
    scratch_types=[
        pltpu.VMEM((NBAND * 8, V), jnp.float32),
        [pltpu.VMEM((SEG,), jnp.int32)] * 2,
        [pltpu.VMEM((SEG,), jnp.int32)] * 2,
        [pltpu.VMEM((TPS, 8, 128), jnp.float32)] * NSTG,
        pltpu.VMEM((V,), jnp.float32),
        pltpu.VMEM((ROWS_PER_TILE,), jnp.int32),
        pltpu.VMEM((L,), jnp.float32),
        [pltpu.SemaphoreType.DMA] * 2,
        [pltpu.SemaphoreType.DMA] * 2,
        [pltpu.SemaphoreType.DMA] * NSTG,
    ],
)
def _sc_gather(tt_hbm, idxf_hbm, tgtf_hbm, lse_hbm,
               out_hbm, part_hbm,
               band_v, idx_segs, tgt_segs, stg_bufs, lse_v, idx_own, part_v,
               isems, tsems, ssems):
    wid = lax.axis_index("s") * NC + lax.axis_index("c")
    base = wid * ROWS_PER_TILE

    band_handles = []
    for b in range(NBAND):
        row0 = (wid + 32 * b) * 8
        row0 = jnp.minimum(row0, V - 8)
        band_handles.append(pltpu.async_copy(
            tt_hbm.at[pl.ds(row0, 8)], band_v.at[pl.ds(b * 8, 8)],
            ssems[b]))
    band_handles.append(pltpu.async_copy(lse_hbm, lse_v, ssems[NBAND]))
    band_handles.append(pltpu.async_copy(
        idxf_hbm.at[pl.ds(base, ROWS_PER_TILE)], idx_own, ssems[NBAND + 1]))

    def start_seg_load(s, p):
        pltpu.async_copy(idxf_hbm.at[pl.ds(s * SEG, SEG)],
                         idx_segs[p], isems[p])
        pltpu.async_copy(tgtf_hbm.at[pl.ds(s * SEG, SEG)],
                         tgt_segs[p], tsems[p])

    def wait_seg_load(s, p):
        pltpu.make_async_copy(idxf_hbm.at[pl.ds(s * SEG, SEG)],
                              idx_segs[p], isems[p]).wait()
        pltpu.make_async_copy(tgtf_hbm.at[pl.ds(s * SEG, SEG)],
                              tgt_segs[p], tsems[p]).wait()

    def start_stg_out(s, bi, vh):
        pltpu.async_copy(stg_bufs[bi], out_hbm.at[vh, pl.ds(s * TPS, TPS)],
                         ssems[bi])

    def wait_stg_out(s, bi, vh):
        pltpu.make_async_copy(stg_bufs[bi],
                              out_hbm.at[vh, pl.ds(s * TPS, TPS)],
                              ssems[bi]).wait()

    start_seg_load(0, 0)
    start_seg_load(1, 1)
    for h in band_handles:
        h.wait()

    def seg_pair(g, acc):
        for half in range(2):
            s = g * 2 + half
            wait_seg_load(s, half)
            idx_seg = idx_segs[half]
            tgt_seg = tgt_segs[half]

            for b in range(NBAND):
                vh = wid + 32 * b
                bi = b + NBAND * half
                stg = stg_bufs[bi]

                def fill_body(q, b=b, stg=stg, idx_seg=idx_seg):
                    t = lax.shift_right_logical(q, 3)
                    k = jnp.bitwise_and(q, 7)
                    col = idx_seg[pl.ds(q * L, L)]
                    for vl in range(8):
                        rows = jnp.full((L,), b * 8 + vl, jnp.int32)
                        gv = plsc.load_gather(band_v, [rows, col])
                        stg[t, vl, pl.ds(k * L, L)] = gv

                def band_work(bi=bi, vh=vh, s=s, fill_body=fill_body):
                    @pl.when(s > 1)
                    def _():
                        wait_stg_out(s - 2, bi, vh)
                    plsc.parallel_loop(0, SEG // L, 1, unroll=16)(fill_body)
                    start_stg_out(s, bi, vh)

                if b == NBAND - 1:
                    pl.when(vh < NVH)(band_work)
                else:
                    band_work()

            @pl.when(s + 2 < NSEG)
            def _():
                start_seg_load(s + 2, half)

            def loss_body(k, acc, idx_seg=idx_seg, tgt_seg=tgt_seg):
                il = idx_seg[pl.ds(k * L, L)]
                tl = tgt_seg[pl.ds(k * L, L)]
                t3 = lax.shift_right_logical(tl, 3)
                t7 = jnp.bitwise_and(tl, 7)
                for b in range(NBAND):
                    sel = t7 + (b * 8)
                    gv = plsc.load_gather(band_v, [sel, il])
                    hit = t3 == (wid + 32 * b)
                    acc = acc - jnp.where(hit, gv,
                                          jnp.zeros((L,), jnp.float32))
                return acc
            acc = plsc.parallel_loop(0, SEG // L, 1, unroll=4,
                                     carry=acc)(loss_body)
        return acc

    acc = lax.fori_loop(0, NSEG // 2, seg_pair, jnp.zeros((L,), jnp.float32))

    def lse_body(g, acc):
        il = idx_own[pl.ds(g * L, L)]
        return acc + plsc.load_gather(lse_v, [il])
    acc = plsc.parallel_loop(0, ROWS_PER_TILE // L, 1, unroll=4,
                             carry=acc)(lse_body)

    for half in range(2):
        s = NSEG - 2 + half
        for b in range(NBAND):
            vh = wid + 32 * b
            bi = b + NBAND * half
            if b == NBAND - 1:
                @pl.when(vh < NVH)
                def _(s=s, bi=bi, vh=vh):
                    wait_stg_out(s, bi, vh)
            else:
                wait_stg_out(s, bi, vh)
    part_v[...] = acc
    pltpu.sync_copy(part_v, part_hbm.at[wid])


def _loss_body(p_ref, out_ref):
    out_ref[...] = jnp.sum(p_ref[...]).reshape(1, 1) / NFLAT


def _loss_reduce(partials):
    return pl.pallas_call(
        _loss_body,
        out_shape=jax.ShapeDtypeStruct((1, 1), jnp.float32),
    )(partials)


def kernel(idx, targets, table):
    idx_flat = idx.astype(jnp.int32).reshape(NFLAT)
    tgt_flat = targets.astype(jnp.int32).reshape(NFLAT)
    table = table.astype(jnp.float32)
    lse, table_t = _table_prep(table)
    out4, partials = _sc_gather(table_t, idx_flat, tgt_flat,
                                lse.reshape(V))
    logits = out4.transpose(1, 3, 0, 2).reshape(NFLAT, V)
    loss = _loss_reduce(partials)[0, 0]
    return logits, loss


# speedup vs baseline: 1.4398x; 1.4398x over previous
import functools

import jax
import jax.numpy as jnp
from jax import lax
from jax.experimental import pallas as pl
from jax.experimental.pallas import tpu as pltpu
from jax.experimental.pallas import tpu_sc as plsc

V = 1000
NFLAT = 51200
NC, NS, L = 2, 16, 16
NW = NC * NS
NVH = V // 8
NBAND = 4
SEG = 1024
NSEG = NFLAT // SEG
TPS = SEG // 128
NSTG = 2 * NBAND
ROWS_PER_TILE = NFLAT // NW


def _lse_body(tab_ref, lse_ref, tt_ref):
    x = tab_ref[...]
    m = jnp.max(x, axis=1, keepdims=True)
    s = jnp.sum(jnp.exp(x - m), axis=1, keepdims=True)
    lse_ref[...] = m + jnp.log(s)
    tt_ref[...] = x.T


def _table_prep(table):
    return pl.pallas_call(
        _lse_body,
        out_shape=[
            jax.ShapeDtypeStruct((V, 1), jnp.float32),
            jax.ShapeDtypeStruct((V, V), jnp.float32),
        ],
    )(table)


_mesh = plsc.VectorSubcoreMesh(core_axis_name="c", subcore_axis_name="s")


@functools.partial(
    pl.kernel,
    mesh=_mesh,
    compiler_params=pltpu.CompilerParams(
        use_tc_tiling_on_sc=False, needs_layout_passes=False),
    out_type=[
        jax.ShapeDtypeStruct((NVH, NFLAT // 128, 8, 128), jnp.float32),
        jax.ShapeDtypeStruct((NW, L), jnp.float32),
    ],
    scratch_types=[
        pltpu.VMEM((NBAND * 8, V), jnp.float32),
        [pltpu.VMEM((SEG,), jnp.int32)] * 2,
        [pltpu.VMEM((SEG,), jnp.int32)] * 2,
        [pltpu.VMEM((TPS, 8, 128), jnp.float32)] * NSTG,
        pltpu.VMEM((V,), jnp.float32),
        pltpu.VMEM((ROWS_PER_TILE,), jnp.int32),
        pltpu.VMEM((L,), jnp.float32),
        [pltpu.SemaphoreType.DMA] * 2,
        [pltpu.SemaphoreType.DMA] * 2,
        [pltpu.SemaphoreType.DMA] * NSTG,
    ],
)
def _sc_gather(tt_hbm, idxf_hbm, tgtf_hbm, lse_hbm,
               out_hbm, part_hbm,
               band_v, idx_segs, tgt_segs, stg_bufs, lse_v, idx_own, part_v,
               isems, tsems, ssems):
    wid = lax.axis_index("s") * NC + lax.axis_index("c")
    base = wid * ROWS_PER_TILE

    band_handles = []
    for b in range(NBAND):
        row0 = (wid + 32 * b) * 8
        row0 = jnp.minimum(row0, V - 8)
        band_handles.append(pltpu.async_copy(
            tt_hbm.at[pl.ds(row0, 8)], band_v.at[pl.ds(b * 8, 8)],
            ssems[b]))
    band_handles.append(pltpu.async_copy(lse_hbm, lse_v, ssems[NBAND]))
    band_handles.append(pltpu.async_copy(
        idxf_hbm.at[pl.ds(base, ROWS_PER_TILE)], idx_own, ssems[NBAND + 1]))

    def start_seg_load(s, p):
        pltpu.async_copy(idxf_hbm.at[pl.ds(s * SEG, SEG)],
                         idx_segs[p], isems[p])
        pltpu.async_copy(tgtf_hbm.at[pl.ds(s * SEG, SEG)],
                         tgt_segs[p], tsems[p])

    def wait_seg_load(s, p):
        pltpu.make_async_copy(idxf_hbm.at[pl.ds(s * SEG, SEG)],
                              idx_segs[p], isems[p]).wait()
        pltpu.make_async_copy(tgtf_hbm.at[pl.ds(s * SEG, SEG)],
                              tgt_segs[p], tsems[p]).wait()

    def start_stg_out(s, bi, vh):
        pltpu.async_copy(stg_bufs[bi], out_hbm.at[vh, pl.ds(s * TPS, TPS)],
                         ssems[bi])

    def wait_stg_out(s, bi, vh):
        pltpu.make_async_copy(stg_bufs[bi],
                              out_hbm.at[vh, pl.ds(s * TPS, TPS)],
                              ssems[bi]).wait()

    start_seg_load(0, 0)
    start_seg_load(1, 1)
    for h in band_handles:
        h.wait()

    def seg_pair(g, acc):
        for half in range(2):
            s = g * 2 + half
            wait_seg_load(s, half)
            idx_seg = idx_segs[half]
            tgt_seg = tgt_segs[half]

            for b in range(NBAND):
                vh = wid + 32 * b
                bi = b + NBAND * half
                stg = stg_bufs[bi]

                def fill_body(q, b=b, stg=stg, idx_seg=idx_seg):
                    t = lax.shift_right_logical(q, 3)
                    k = jnp.bitwise_and(q, 7)
                    col = idx_seg[pl.ds(q * L, L)]
                    for vl in range(8):
                        rows = jnp.full((L,), b * 8 + vl, jnp.int32)
                        gv = plsc.load_gather(band_v, [rows, col])
                        stg[t, vl, pl.ds(k * L, L)] = gv

                def band_work(bi=bi, vh=vh, s=s, fill_body=fill_body):
                    @pl.when(s > 1)
                    def _():
                        wait_stg_out(s - 2, bi, vh)
                    plsc.parallel_loop(0, SEG // L, 1, unroll=8)(fill_body)
                    start_stg_out(s, bi, vh)

                if b == NBAND - 1:
                    pl.when(vh < NVH)(band_work)
                else:
                    band_work()

            @pl.when(s + 2 < NSEG)
            def _():
                start_seg_load(s + 2, half)

            def loss_body(k, acc, idx_seg=idx_seg, tgt_seg=tgt_seg):
                il = idx_seg[pl.ds(k * L, L)]
                tl = tgt_seg[pl.ds(k * L, L)]
                t3 = lax.shift_right_logical(tl, 3)
                t7 = jnp.bitwise_and(tl, 7)
                for b in range(NBAND):
                    sel = t7 + (b * 8)
                    gv = plsc.load_gather(band_v, [sel, il])
                    hit = t3 == (wid + 32 * b)
                    acc = acc - jnp.where(hit, gv,
                                          jnp.zeros((L,), jnp.float32))
                return acc
            acc = plsc.parallel_loop(0, SEG // L, 1, unroll=4,
                                     carry=acc)(loss_body)
        return acc

    acc = lax.fori_loop(0, NSEG // 2, seg_pair, jnp.zeros((L,), jnp.float32))

    def lse_body(g, acc):
        il = idx_own[pl.ds(g * L, L)]
        return acc + plsc.load_gather(lse_v, [il])
    acc = plsc.parallel_loop(0, ROWS_PER_TILE // L, 1, unroll=4,
                             carry=acc)(lse_body)

    for half in range(2):
        s = NSEG - 2 + half
        for b in range(NBAND):
            vh = wid + 32 * b
            bi = b + NBAND * half
            if b == NBAND - 1:
                @pl.when(vh < NVH)
                def _(s=s, bi=bi, vh=vh):
                    wait_stg_out(s, bi, vh)
            else:
                wait_stg_out(s, bi, vh)
    part_v[...] = acc
    pltpu.sync_copy(part_v, part_hbm.at[wid])


def _loss_body(p_ref, out_ref):
    out_ref[...] = jnp.sum(p_ref[...]).reshape(1, 1) / NFLAT


def _loss_reduce(partials):
    return pl.pallas_call(
        _loss_body,
        out_shape=jax.ShapeDtypeStruct((1, 1), jnp.float32),
    )(partials)


def kernel(idx, targets, table):
    idx_flat = idx.astype(jnp.int32).reshape(NFLAT)
    tgt_flat = targets.astype(jnp.int32).reshape(NFLAT)
    table = table.astype(jnp.float32)
    lse, table_t = _table_prep(table)
    out4, partials = _sc_gather(table_t, idx_flat, tgt_flat,
                                lse.reshape(V))
    logits = out4.transpose(1, 3, 0, 2).reshape(NFLAT, V)
    loss = _loss_reduce(partials)[0, 0]
    return logits, loss
